# baseline (speedup 1.0000x reference)
import functools

import jax
import jax.numpy as jnp
from jax import lax
from jax.experimental import pallas as pl
from jax.experimental.pallas import tpu as pltpu
from jax.experimental.pallas import tpu_sc as plsc

N = 10000
NR = 3
TN = NR * N
RPAD = TN + 80
E = 320000
NC = 2
NS = 16
NW = NC * NS
CHUNK = 128
NCH0 = 88
NCH1 = 72
NCHT = NS * (NCH0 + NCH1)
EPAD = NCHT * CHUNK
RPS = RPAD // NS
HALF = N // 2
D = 32
NB = 2000
_f32 = jnp.float32


def _edge_body(table, gidx, sidx, zrows, out, gi_all, si_all, rows_bf,
               rows, acc_sh, gsem, ssem):
    cid = lax.axis_index("c")
    sid = lax.axis_index("s")
    r0 = sid * RPS
    pltpu.sync_copy(zrows.at[pl.ds(r0, RPS)], acc_sh.at[pl.ds(r0, RPS)])

    def _wait_gather(b):
        pltpu.make_async_copy(table.at[gi_all.at[0]], rows_bf.at[b],
                              gsem).wait()

    def _wait_scatter(b):
        pltpu.make_async_copy(zrows.at[pl.ds(0, CHUNK)], rows.at[b],
                              ssem).wait()

    def _convert(b):
        def cv(i, carry):
            for j in range(16):
                rr = i * 16 + j
                w = plsc.bitcast(rows_bf[b, rr, :], jnp.int32)
                rows[b, rr, 0:16] = plsc.bitcast(w << 16, jnp.float32)
                rows[b, rr, 16:32] = plsc.bitcast(
                    jnp.bitwise_and(w, jnp.int32(-65536)), jnp.float32)
            return carry
        lax.fori_loop(0, CHUNK // 16, cv, 0)

    def _run(nch, cbase):
        pltpu.sync_copy(gidx.at[pl.ds(cbase, nch)], gi_all.at[pl.ds(0, nch)])
        pltpu.sync_copy(sidx.at[pl.ds(cbase, nch)], si_all.at[pl.ds(0, nch)])
        plsc.subcore_barrier()

        for b in range(2):
            pltpu.async_copy(table.at[gi_all.at[b]], rows_bf.at[b], gsem)

        def outer(k, carry):
            for b in range(4):
                c = 4 * k + b
                _wait_gather(b)
                bn = (b + 2) % 4
                @pl.when((c + 2) < nch)
                def _():
                    pltpu.async_copy(table.at[gi_all.at[c + 2]],
                                     rows_bf.at[bn], gsem)
                @pl.when((c - 2) >= 0)
                def _():
                    _wait_scatter(b)
                _convert(b)
                pltpu.async_copy(rows.at[b], acc_sh.at[si_all.at[c]], ssem,
                                 add=True)
            return carry

        lax.fori_loop(0, nch // 4, outer, 0)
        _wait_scatter((nch - 2) % 4)
        _wait_scatter((nch - 1) % 4)

        plsc.subcore_barrier()
        pltpu.sync_copy(acc_sh.at[pl.ds(r0, RPS)], out.at[cid, pl.ds(r0, RPS)])

    @pl.when(cid == 0)
    def _():
        _run(NCH0, sid * NCH0)

    @pl.when(cid == 1)
    def _():
        _run(NCH1, NS * NCH0 + sid * NCH1)


def _make_edge_call(width):
    mesh = plsc.VectorSubcoreMesh(core_axis_name="c", subcore_axis_name="s")
    return pl.kernel(
        _edge_body,
        out_type=jax.ShapeDtypeStruct((NC, RPAD, width), jnp.float32),
        mesh=mesh,
        scratch_types=[
            pltpu.VMEM((NCH0, CHUNK), jnp.int32),
            pltpu.VMEM((NCH0, CHUNK), jnp.int32),
            pltpu.VMEM((4, CHUNK, width), jnp.bfloat16),
            pltpu.VMEM((4, CHUNK, width), jnp.float32),
            pltpu.VMEM_SHARED((RPAD, width), jnp.float32),
            pltpu.SemaphoreType.DMA,
            pltpu.SemaphoreType.DMA,
        ],
        compiler_params=pltpu.CompilerParams(use_tc_tiling_on_sc=False,
                                             needs_layout_passes=False),
        name=f"edge_agg_{width}",
    )


_edge32 = _make_edge_call(D)


def _deg_body(sidx, ones, zrows, out, si_all, ones_v, acc_sh, ssem):
    cid = lax.axis_index("c")
    sid = lax.axis_index("s")
    r0 = sid * RPS
    pltpu.sync_copy(zrows.at[pl.ds(r0, RPS)], acc_sh.at[pl.ds(r0, RPS)])
    pltpu.sync_copy(ones, ones_v)

    def _wait_scatter():
        pltpu.make_async_copy(zrows.at[pl.ds(0, CHUNK)], ones_v, ssem).wait()

    LAG = 4

    def _run(nch, cbase):
        pltpu.sync_copy(sidx.at[pl.ds(cbase, nch)], si_all.at[pl.ds(0, nch)])
        plsc.subcore_barrier()

        def step(c, carry):
            pltpu.async_copy(ones_v, acc_sh.at[si_all.at[c]], ssem, add=True)
            @pl.when(c >= LAG)
            def _():
                _wait_scatter()
            return carry

        lax.fori_loop(0, nch, step, 0)
        for _ in range(LAG):
            _wait_scatter()
        plsc.subcore_barrier()
        pltpu.sync_copy(acc_sh.at[pl.ds(r0, RPS)], out.at[cid, pl.ds(r0, RPS)])

    @pl.when(cid == 0)
    def _():
        _run(NCH0, sid * NCH0)

    @pl.when(cid == 1)
    def _():
        _run(NCH1, NS * NCH0 + sid * NCH1)


_deg = pl.kernel(
    _deg_body,
    out_type=jax.ShapeDtypeStruct((NC, RPAD, D), jnp.float32),
    mesh=plsc.VectorSubcoreMesh(core_axis_name="c", subcore_axis_name="s"),
    scratch_types=[
        pltpu.VMEM((NCH0, CHUNK), jnp.int32),
        pltpu.VMEM((CHUNK, D), jnp.float32),
        pltpu.VMEM_SHARED((RPAD, D), jnp.float32),
        pltpu.SemaphoreType.DMA,
    ],
    compiler_params=pltpu.CompilerParams(use_tc_tiling_on_sc=False),
    name="deg_count",
)


def _rel_weights(b_ref, c_ref):
    bs = b_ref[...]
    cm = c_ref[...]
    return [cm[r:r + 1, 0:1] * bs[0] + cm[r:r + 1, 1:2] * bs[1]
            for r in range(NR)]


PK = RPAD * D // 128
PKR = N * D // 128
PH = HALF // 4


def _bd4(w):
    a, b = w.shape
    rows = []
    for j in range(4):
        blocks = [jnp.zeros((a, b * j), _f32), w,
                  jnp.zeros((a, b * (3 - j)), _f32)]
        rows.append(jnp.concatenate([x for x in blocks if x.shape[1]], axis=1))
    return jnp.concatenate(rows, axis=0)


def _seg_expand(seg):
    a, b = seg.shape
    rows = []
    for j in range(4):
        blocks = [jnp.zeros((a, b * j), _f32), seg,
                  jnp.zeros((a, b * (3 - j)), _f32)]
        rows.append(jnp.concatenate([x for x in blocks if x.shape[1]], axis=1))
    return jnp.concatenate(rows, axis=0)


def _tile4(v):
    return jnp.concatenate([v, v, v, v], axis=1)


def _interleave_cols(w):
    return jnp.concatenate(
        [w[:, (m // 2) + 16 * (m % 2):(m // 2) + 16 * (m % 2) + 1]
         for m in range(D)], axis=1)


def _write_tab(tab_ref, hp, ws):
    for r in range(NR):
        tab_ref[r * PKR:(r + 1) * PKR, :] = jnp.dot(
            hp, _bd4(_interleave_cols(ws[r])),
            preferred_element_type=_f32).astype(jnp.bfloat16)
    tab_ref[NR * PKR:, :] = jnp.zeros((PK - NR * PKR, 128), jnp.bfloat16)


def _agg(acc_ref, r):
    return (acc_ref[0, r * PKR:(r + 1) * PKR, :]
            + acc_ref[1, r * PKR:(r + 1) * PKR, :])


def _pre0_body(x4_ref, b_ref, c_ref, rt_ref, bi_ref, tab_ref, rv_ref):
    x4 = x4_ref[...]
    _write_tab(tab_ref, x4, _rel_weights(b_ref, c_ref))
    rv_ref[...] = jnp.dot(x4, _bd4(rt_ref[...]),
                          preferred_element_type=_f32) + _tile4(bi_ref[...])


def _fcomb0_body(acc_ref, dacc_ref, rv_ref, b_ref, c_ref, rt_ref, bi_ref,
                 h_ref, tab_ref, rvn_ref, dinv_ref):
    out = rv_ref[...]
    for r in range(NR):
        dblk = (dacc_ref[0, r * PKR:(r + 1) * PKR, :]
                + dacc_ref[1, r * PKR:(r + 1) * PKR, :])
        dinv = 1.0 / jnp.maximum(dblk, 1.0)
        dinv_ref[r * PKR:(r + 1) * PKR, :] = dinv
        out = out + _agg(acc_ref, r) * dinv
    h = jnp.tanh(out)
    h_ref[...] = h
    _write_tab(tab_ref, h, _rel_weights(b_ref, c_ref))
    rvn_ref[...] = jnp.dot(h, _bd4(rt_ref[...]),
                           preferred_element_type=_f32) + _tile4(bi_ref[...])


def _fcomb_body(acc_ref, rv_ref, dinv_ref, b_ref, c_ref, rt_ref, bi_ref,
                h_ref, tab_ref, rvn_ref):
    out = rv_ref[...]
    for r in range(NR):
        out = out + _agg(acc_ref, r) * dinv_ref[r * PKR:(r + 1) * PKR, :]
    h = jnp.tanh(out)
    h_ref[...] = h
    _write_tab(tab_ref, h, _rel_weights(b_ref, c_ref))
    rvn_ref[...] = jnp.dot(h, _bd4(rt_ref[...]),
                           preferred_element_type=_f32) + _tile4(bi_ref[...])


def _fhead_body(acc_ref, rv_ref, dinv_ref, h0_ref, h1_ref, h2_ref,
                w1_ref, b1_ref, w2_ref, b2_ref, out_ref):
    out = rv_ref[...]
    for r in range(NR):
        out = out + _agg(acc_ref, r) * dinv_ref[r * PKR:(r + 1) * PKR, :]
    h3 = jnp.tanh(out)
    w1 = w1_ref[...]
    hs = [h0_ref[...], h1_ref[...], h2_ref[...], h3]
    z1 = _tile4(b1_ref[...])
    for l in range(4):
        hu = hs[l][PH:, :]
        hb = hs[l][:PH, :]
        z1 = z1 + jnp.dot(hu, _seg_expand(w1[32 * l:32 * l + 32, :]),
                          preferred_element_type=_f32)
        z1 = z1 + jnp.dot(hb, _seg_expand(w1[128 + 32 * l:128 + 32 * l + 32, :]),
                          preferred_element_type=_f32)
    z1 = jnp.maximum(z1, 0.0)
    z2 = jnp.dot(z1, _bd4(w2_ref[...]),
                 preferred_element_type=_f32) + _tile4(b2_ref[...])
    out_ref[...] = jax.nn.sigmoid(z2)


_pre0 = pl.pallas_call(
    _pre0_body,
    out_shape=(jax.ShapeDtypeStruct((PK, 128), jnp.bfloat16),
               jax.ShapeDtypeStruct((PKR, 128), _f32)),
)
_fcomb0 = pl.pallas_call(
    _fcomb0_body,
    out_shape=(jax.ShapeDtypeStruct((PKR, 128), _f32),
               jax.ShapeDtypeStruct((PK, 128), jnp.bfloat16),
               jax.ShapeDtypeStruct((PKR, 128), _f32),
               jax.ShapeDtypeStruct((NR * PKR, 128), _f32)),
)
_fcomb = pl.pallas_call(
    _fcomb_body,
    out_shape=(jax.ShapeDtypeStruct((PKR, 128), _f32),
               jax.ShapeDtypeStruct((PK, 128), jnp.bfloat16),
               jax.ShapeDtypeStruct((PKR, 128), _f32)),
)
_fhead = pl.pallas_call(
    _fhead_body,
    out_shape=jax.ShapeDtypeStruct((PH, 4), _f32),
)


def kernel(x, edge_index, edge_type, bases0, comp0, root0, bias0,
           bases1, comp1, root1, bias1, bases2, comp2, root2, bias2,
           bases3, comp3, root3, bias3, w1, b1, w2, b2):
    src, dst = edge_index[0], edge_index[1]
    pad = jnp.full((EPAD - E,), TN, jnp.int32)
    gidx = jnp.concatenate([edge_type * N + src, pad]).reshape(EPAD // CHUNK, CHUNK)
    sidx = jnp.concatenate([edge_type * N + dst, pad]).reshape(EPAD // CHUNK, CHUNK)
    z32 = jnp.zeros((RPAD, D), _f32)
    ones32 = jnp.ones((CHUNK, D), _f32)
    x4 = x.reshape(PKR, 4 * 128)

    dacc = _deg(sidx, ones32, z32).reshape(NC, PK, 128)
    tab0p, rv0 = _pre0(x4, bases0, comp0, root0, bias0.reshape(1, D))
    acc0 = _edge32(tab0p.reshape(RPAD, D), gidx, sidx, z32).reshape(NC, PK, 128)
    h0, tab1p, rv1, dinv = _fcomb0(acc0, dacc, rv0, bases1, comp1, root1,
                                   bias1.reshape(1, D))

    acc1 = _edge32(tab1p.reshape(RPAD, D), gidx, sidx, z32).reshape(NC, PK, 128)
    h1, tab2p, rv2 = _fcomb(acc1, rv1, dinv, bases2, comp2, root2,
                            bias2.reshape(1, D))

    acc2 = _edge32(tab2p.reshape(RPAD, D), gidx, sidx, z32).reshape(NC, PK, 128)
    h2, tab3p, rv3 = _fcomb(acc2, rv2, dinv, bases3, comp3, root3,
                            bias3.reshape(1, D))

    acc3 = _edge32(tab3p.reshape(RPAD, D), gidx, sidx, z32).reshape(NC, PK, 128)
    out = _fhead(acc3, rv3, dinv, h0, h1, h2,
                 w1, b1.reshape(1, 128), w2, b2.reshape(1, 1))
    return out.reshape(HALF)


# speedup vs baseline: 1.0247x; 1.0247x over previous
import functools

import jax
import jax.numpy as jnp
from jax import lax
from jax.experimental import pallas as pl
from jax.experimental.pallas import tpu as pltpu
from jax.experimental.pallas import tpu_sc as plsc

N = 10000
NR = 3
TN = NR * N
RPAD = TN + 80
E = 320000
NC = 2
NS = 16
NW = NC * NS
CHUNK = 128
NCH0 = 88
NCH1 = 72
NCHT = NS * (NCH0 + NCH1)
EPAD = NCHT * CHUNK
RPS = RPAD // NS
HALF = N // 2
D = 32
NB = 2000
_f32 = jnp.float32


def _edge_body(table, gidx, sidx, zrows, out, gi_all, si_all, rows_bf,
               rows, acc_sh, gsem, ssem):
    cid = lax.axis_index("c")
    sid = lax.axis_index("s")
    r0 = sid * RPS
    pltpu.sync_copy(zrows.at[pl.ds(r0, RPS)], acc_sh.at[pl.ds(r0, RPS)])

    def _wait_gather(b):
        pltpu.make_async_copy(table.at[gi_all.at[0]], rows_bf.at[b],
                              gsem).wait()

    def _wait_scatter(b):
        pltpu.make_async_copy(zrows.at[pl.ds(0, CHUNK)], rows.at[b],
                              ssem).wait()

    def _convert(b):
        def cv(i, carry):
            for j in range(8):
                rr = i * 8 + j
                w = plsc.bitcast(rows_bf[b, rr, :], jnp.int32)
                rows[b, rr, 0:16] = plsc.bitcast(w << 16, jnp.float32)
                rows[b, rr, 16:32] = plsc.bitcast(
                    jnp.bitwise_and(w, jnp.int32(-65536)), jnp.float32)
            return carry
        lax.fori_loop(0, CHUNK // 8, cv, 0)

    def _run(nch, cbase):
        pltpu.sync_copy(gidx.at[pl.ds(cbase, nch)], gi_all.at[pl.ds(0, nch)])
        pltpu.sync_copy(sidx.at[pl.ds(cbase, nch)], si_all.at[pl.ds(0, nch)])
        plsc.subcore_barrier()

        for b in range(2):
            pltpu.async_copy(table.at[gi_all.at[b]], rows_bf.at[b], gsem)

        def outer(k, carry):
            for b in range(4):
                c = 4 * k + b
                _wait_gather(b)
                bn = (b + 2) % 4
                @pl.when((c + 2) < nch)
                def _():
                    pltpu.async_copy(table.at[gi_all.at[c + 2]],
                                     rows_bf.at[bn], gsem)
                @pl.when((c - 2) >= 0)
                def _():
                    _wait_scatter(b)
                _convert(b)
                pltpu.async_copy(rows.at[b], acc_sh.at[si_all.at[c]], ssem,
                                 add=True)
            return carry

        lax.fori_loop(0, nch // 4, outer, 0)
        _wait_scatter((nch - 2) % 4)
        _wait_scatter((nch - 1) % 4)

        plsc.subcore_barrier()
        pltpu.sync_copy(acc_sh.at[pl.ds(r0, RPS)], out.at[cid, pl.ds(r0, RPS)])

    @pl.when(cid == 0)
    def _():
        _run(NCH0, sid * NCH0)

    @pl.when(cid == 1)
    def _():
        _run(NCH1, NS * NCH0 + sid * NCH1)


def _make_edge_call(width):
    mesh = plsc.VectorSubcoreMesh(core_axis_name="c", subcore_axis_name="s")
    return pl.kernel(
        _edge_body,
        out_type=jax.ShapeDtypeStruct((NC, RPAD, width), jnp.float32),
        mesh=mesh,
        scratch_types=[
            pltpu.VMEM((NCH0, CHUNK), jnp.int32),
            pltpu.VMEM((NCH0, CHUNK), jnp.int32),
            pltpu.VMEM((4, CHUNK, width), jnp.bfloat16),
            pltpu.VMEM((4, CHUNK, width), jnp.float32),
            pltpu.VMEM_SHARED((RPAD, width), jnp.float32),
            pltpu.SemaphoreType.DMA,
            pltpu.SemaphoreType.DMA,
        ],
        compiler_params=pltpu.CompilerParams(use_tc_tiling_on_sc=False,
                                             needs_layout_passes=False),
        name=f"edge_agg_{width}",
    )


_edge32 = _make_edge_call(D)


def _deg_body(sidx, ones, zrows, out, si_all, ones_v, acc_sh, ssem):
    cid = lax.axis_index("c")
    sid = lax.axis_index("s")
    r0 = sid * RPS
    pltpu.sync_copy(zrows.at[pl.ds(r0, RPS)], acc_sh.at[pl.ds(r0, RPS)])
    pltpu.sync_copy(ones, ones_v)

    def _wait_scatter():
        pltpu.make_async_copy(zrows.at[pl.ds(0, CHUNK)], ones_v, ssem).wait()

    LAG = 4

    def _run(nch, cbase):
        pltpu.sync_copy(sidx.at[pl.ds(cbase, nch)], si_all.at[pl.ds(0, nch)])
        plsc.subcore_barrier()

        def step(c, carry):
            pltpu.async_copy(ones_v, acc_sh.at[si_all.at[c]], ssem, add=True)
            @pl.when(c >= LAG)
            def _():
                _wait_scatter()
            return carry

        lax.fori_loop(0, nch, step, 0)
        for _ in range(LAG):
            _wait_scatter()
        plsc.subcore_barrier()
        pltpu.sync_copy(acc_sh.at[pl.ds(r0, RPS)], out.at[cid, pl.ds(r0, RPS)])

    @pl.when(cid == 0)
    def _():
        _run(NCH0, sid * NCH0)

    @pl.when(cid == 1)
    def _():
        _run(NCH1, NS * NCH0 + sid * NCH1)


_deg = pl.kernel(
    _deg_body,
    out_type=jax.ShapeDtypeStruct((NC, RPAD, D), jnp.float32),
    mesh=plsc.VectorSubcoreMesh(core_axis_name="c", subcore_axis_name="s"),
    scratch_types=[
        pltpu.VMEM((NCH0, CHUNK), jnp.int32),
        pltpu.VMEM((CHUNK, D), jnp.float32),
        pltpu.VMEM_SHARED((RPAD, D), jnp.float32),
        pltpu.SemaphoreType.DMA,
    ],
    compiler_params=pltpu.CompilerParams(use_tc_tiling_on_sc=False),
    name="deg_count",
)


def _rel_weights(b_ref, c_ref):
    bs = b_ref[...]
    cm = c_ref[...]
    return [cm[r:r + 1, 0:1] * bs[0] + cm[r:r + 1, 1:2] * bs[1]
            for r in range(NR)]


PK = RPAD * D // 128
PKR = N * D // 128
PH = HALF // 4


def _bd4(w):
    a, b = w.shape
    rows = []
    for j in range(4):
        blocks = [jnp.zeros((a, b * j), _f32), w,
                  jnp.zeros((a, b * (3 - j)), _f32)]
        rows.append(jnp.concatenate([x for x in blocks if x.shape[1]], axis=1))
    return jnp.concatenate(rows, axis=0)


def _seg_expand(seg):
    a, b = seg.shape
    rows = []
    for j in range(4):
        blocks = [jnp.zeros((a, b * j), _f32), seg,
                  jnp.zeros((a, b * (3 - j)), _f32)]
        rows.append(jnp.concatenate([x for x in blocks if x.shape[1]], axis=1))
    return jnp.concatenate(rows, axis=0)


def _tile4(v):
    return jnp.concatenate([v, v, v, v], axis=1)


def _interleave_cols(w):
    return jnp.concatenate(
        [w[:, (m // 2) + 16 * (m % 2):(m // 2) + 16 * (m % 2) + 1]
         for m in range(D)], axis=1)


def _write_tab(tab_ref, hp, ws):
    for r in range(NR):
        tab_ref[r * PKR:(r + 1) * PKR, :] = jnp.dot(
            hp, _bd4(_interleave_cols(ws[r])), preferred_element_type=_f32)
    tab_ref[NR * PKR:, :] = jnp.zeros((PK - NR * PKR, 128), _f32)


def _agg(acc_ref, r):
    return (acc_ref[0, r * PKR:(r + 1) * PKR, :]
            + acc_ref[1, r * PKR:(r + 1) * PKR, :])


def _pre0_body(x4_ref, b_ref, c_ref, rt_ref, bi_ref, tab_ref, rv_ref):
    x4 = x4_ref[...]
    _write_tab(tab_ref, x4, _rel_weights(b_ref, c_ref))
    rv_ref[...] = jnp.dot(x4, _bd4(rt_ref[...]),
                          preferred_element_type=_f32) + _tile4(bi_ref[...])


def _fcomb0_body(acc_ref, dacc_ref, rv_ref, b_ref, c_ref, rt_ref, bi_ref,
                 h_ref, tab_ref, rvn_ref, dinv_ref):
    out = rv_ref[...]
    for r in range(NR):
        dblk = (dacc_ref[0, r * PKR:(r + 1) * PKR, :]
                + dacc_ref[1, r * PKR:(r + 1) * PKR, :])
        dinv = 1.0 / jnp.maximum(dblk, 1.0)
        dinv_ref[r * PKR:(r + 1) * PKR, :] = dinv
        out = out + _agg(acc_ref, r) * dinv
    h = jnp.tanh(out)
    h_ref[...] = h
    _write_tab(tab_ref, h, _rel_weights(b_ref, c_ref))
    rvn_ref[...] = jnp.dot(h, _bd4(rt_ref[...]),
                           preferred_element_type=_f32) + _tile4(bi_ref[...])


def _fcomb_body(acc_ref, rv_ref, dinv_ref, b_ref, c_ref, rt_ref, bi_ref,
                h_ref, tab_ref, rvn_ref):
    out = rv_ref[...]
    for r in range(NR):
        out = out + _agg(acc_ref, r) * dinv_ref[r * PKR:(r + 1) * PKR, :]
    h = jnp.tanh(out)
    h_ref[...] = h
    _write_tab(tab_ref, h, _rel_weights(b_ref, c_ref))
    rvn_ref[...] = jnp.dot(h, _bd4(rt_ref[...]),
                           preferred_element_type=_f32) + _tile4(bi_ref[...])


def _fhead_body(acc_ref, rv_ref, dinv_ref, h0_ref, h1_ref, h2_ref,
                w1_ref, b1_ref, w2_ref, b2_ref, out_ref):
    out = rv_ref[...]
    for r in range(NR):
        out = out + _agg(acc_ref, r) * dinv_ref[r * PKR:(r + 1) * PKR, :]
    h3 = jnp.tanh(out)
    w1 = w1_ref[...]
    hs = [h0_ref[...], h1_ref[...], h2_ref[...], h3]
    z1 = _tile4(b1_ref[...])
    for l in range(4):
        hu = hs[l][PH:, :]
        hb = hs[l][:PH, :]
        z1 = z1 + jnp.dot(hu, _seg_expand(w1[32 * l:32 * l + 32, :]),
                          preferred_element_type=_f32)
        z1 = z1 + jnp.dot(hb, _seg_expand(w1[128 + 32 * l:128 + 32 * l + 32, :]),
                          preferred_element_type=_f32)
    z1 = jnp.maximum(z1, 0.0)
    z2 = jnp.dot(z1, _bd4(w2_ref[...]),
                 preferred_element_type=_f32) + _tile4(b2_ref[...])
    out_ref[...] = jax.nn.sigmoid(z2)


_pre0 = pl.pallas_call(
    _pre0_body,
    out_shape=(jax.ShapeDtypeStruct((PK, 128), _f32),
               jax.ShapeDtypeStruct((PKR, 128), _f32)),
)
_fcomb0 = pl.pallas_call(
    _fcomb0_body,
    out_shape=(jax.ShapeDtypeStruct((PKR, 128), _f32),
               jax.ShapeDtypeStruct((PK, 128), _f32),
               jax.ShapeDtypeStruct((PKR, 128), _f32),
               jax.ShapeDtypeStruct((NR * PKR, 128), _f32)),
)
_fcomb = pl.pallas_call(
    _fcomb_body,
    out_shape=(jax.ShapeDtypeStruct((PKR, 128), _f32),
               jax.ShapeDtypeStruct((PK, 128), _f32),
               jax.ShapeDtypeStruct((PKR, 128), _f32)),
)
_fhead = pl.pallas_call(
    _fhead_body,
    out_shape=jax.ShapeDtypeStruct((PH, 4), _f32),
)


def kernel(x, edge_index, edge_type, bases0, comp0, root0, bias0,
           bases1, comp1, root1, bias1, bases2, comp2, root2, bias2,
           bases3, comp3, root3, bias3, w1, b1, w2, b2):
    src, dst = edge_index[0], edge_index[1]
    pad = jnp.full((EPAD - E,), TN, jnp.int32)
    gidx = jnp.concatenate([edge_type * N + src, pad]).reshape(EPAD // CHUNK, CHUNK)
    sidx = jnp.concatenate([edge_type * N + dst, pad]).reshape(EPAD // CHUNK, CHUNK)
    z32 = jnp.zeros((RPAD, D), _f32)
    ones32 = jnp.ones((CHUNK, D), _f32)
    x4 = x.reshape(PKR, 4 * 128)

    dacc = _deg(sidx, ones32, z32).reshape(NC, PK, 128)
    tab0p, rv0 = _pre0(x4, bases0, comp0, root0, bias0.reshape(1, D))
    acc0 = _edge32(tab0p.astype(jnp.bfloat16).reshape(RPAD, D), gidx, sidx, z32).reshape(NC, PK, 128)
    h0, tab1p, rv1, dinv = _fcomb0(acc0, dacc, rv0, bases1, comp1, root1,
                                   bias1.reshape(1, D))

    acc1 = _edge32(tab1p.astype(jnp.bfloat16).reshape(RPAD, D), gidx, sidx, z32).reshape(NC, PK, 128)
    h1, tab2p, rv2 = _fcomb(acc1, rv1, dinv, bases2, comp2, root2,
                            bias2.reshape(1, D))

    acc2 = _edge32(tab2p.astype(jnp.bfloat16).reshape(RPAD, D), gidx, sidx, z32).reshape(NC, PK, 128)
    h2, tab3p, rv3 = _fcomb(acc2, rv2, dinv, bases3, comp3, root3,
                            bias3.reshape(1, D))

    acc3 = _edge32(tab3p.astype(jnp.bfloat16).reshape(RPAD, D), gidx, sidx, z32).reshape(NC, PK, 128)
    out = _fhead(acc3, rv3, dinv, h0, h1, h2,
                 w1, b1.reshape(1, 128), w2, b2.reshape(1, 1))
    return out.reshape(HALF)
